# baseline (speedup 1.0000x reference)
import functools

import jax
import jax.numpy as jnp
from jax import lax
from jax.experimental import pallas as pl
from jax.experimental.pallas import tpu as pltpu
from jax.experimental.pallas import tpu_sc as plsc

N = 10000
E = 320000
IN_DIM = 128
HID = 64
OUT_DIM = 128

NC = 2
NS = 16
CHUNK = 80
EDGES_PER_TILE = E // (NC * NS)
STEPS = EDGES_PER_TILE // CHUNK
NBUF = 5
RCHUNK = 200
NRCH = N // RCHUNK
NREP = -(-NRCH // NS)
TCHUNK = 80
NTCH = N // TCHUNK
NTREP = -(-NTCH // NS)

_MESH = plsc.VectorSubcoreMesh(core_axis_name="c", subcore_axis_name="s")
_SC_PARAMS = pltpu.CompilerParams(use_tc_tiling_on_sc=False)


def _each_chunk(s, fn):
    for rep in range(NREP):
        ck = s + NS * rep

        @pl.when(ck < NRCH)
        def _():
            fn(ck * RCHUNK)


@functools.partial(
    pl.kernel,
    mesh=_MESH,
    compiler_params=_SC_PARAMS,
    out_type=jax.ShapeDtypeStruct((NC * N,), jnp.float32),
    scratch_types=[
        pltpu.VMEM((STEPS, CHUNK), jnp.int32),
        pltpu.VMEM((CHUNK,), jnp.float32),
        pltpu.VMEM((RCHUNK,), jnp.float32),
        pltpu.VMEM_SHARED((N,), jnp.float32),
    ],
)
def _deg_partials(dst_hbm, out_hbm, idx_v, ones_v, stage_v, deg_sh):
    c = lax.axis_index("c")
    s = lax.axis_index("s")
    tile_row = (c * NS + s) * STEPS
    pltpu.sync_copy(dst_hbm.at[pl.ds(tile_row, STEPS)], idx_v)
    for i in range(CHUNK // 16):
        ones_v[pl.ds(i * 16, 16)] = jnp.ones((16,), jnp.float32)
    for i in range(RCHUNK // 16):
        stage_v[pl.ds(i * 16, 16)] = jnp.ones((16,), jnp.float32)

    def init(r0):
        pltpu.sync_copy(stage_v, deg_sh.at[pl.ds(r0, RCHUNK)])

    _each_chunk(s, init)
    plsc.subcore_barrier()

    def body(i, carry):
        pltpu.sync_copy(ones_v, deg_sh.at[idx_v.at[i]], add=True)
        return carry

    lax.fori_loop(0, STEPS, body, 0)
    plsc.subcore_barrier()

    def writeback(r0):
        pltpu.sync_copy(deg_sh.at[pl.ds(r0, RCHUNK)], stage_v)
        pltpu.sync_copy(stage_v, out_hbm.at[pl.ds(c * N + r0, RCHUNK)])

    _each_chunk(s, writeback)


@functools.partial(
    pl.kernel,
    mesh=_MESH,
    compiler_params=_SC_PARAMS,
    out_type=jax.ShapeDtypeStruct((NC, N, HID), jnp.float32),
    scratch_types=[
        pltpu.VMEM((STEPS, CHUNK), jnp.int32),
        pltpu.VMEM((STEPS, CHUNK), jnp.int32),
        [pltpu.VMEM((CHUNK, HID), jnp.float32)] * NBUF,
        pltpu.VMEM_SHARED((N, HID), jnp.float32),
        pltpu.VMEM_SHARED((N, HID), jnp.float32),
        [pltpu.SemaphoreType.DMA] * NBUF,
    ],
)
def _agg_partials(h_hbm, src_hbm, dst_hbm, out_hbm,
                  sidx_v, didx_v, rows, tab_sh, acc_sh, sems):
    c = lax.axis_index("c")
    s = lax.axis_index("s")
    def stage(r0):
        pltpu.sync_copy(h_hbm.at[pl.ds(r0, TCHUNK)], rows[0])
        pltpu.sync_copy(rows[0], tab_sh.at[pl.ds(r0, TCHUNK)])
        pltpu.sync_copy(rows[0], acc_sh.at[pl.ds(r0, TCHUNK)])

    for rep in range(NTREP):
        ck = s + NS * rep

        @pl.when(ck < NTCH)
        def _():
            stage(ck * TCHUNK)

    tile_row = (c * NS + s) * STEPS
    pltpu.sync_copy(src_hbm.at[pl.ds(tile_row, STEPS)], sidx_v)
    pltpu.sync_copy(dst_hbm.at[pl.ds(tile_row, STEPS)], didx_v)
    plsc.subcore_barrier()

    for b in range(NBUF):
        pltpu.async_copy(tab_sh.at[sidx_v.at[b]], rows[b], sems[b])

    def body(g, carry):
        j0 = g * NBUF
        for b in range(NBUF):
            jj = j0 + b
            pltpu.make_async_copy(tab_sh.at[sidx_v.at[0]], rows[b],
                                  sems[b]).wait()
            pltpu.sync_copy(rows[b], acc_sh.at[didx_v.at[jj]], add=True)

            @pl.when(jj + NBUF < STEPS)
            def _():
                pltpu.async_copy(tab_sh.at[sidx_v.at[jj + NBUF]],
                                 rows[b], sems[b])

        return carry

    lax.fori_loop(0, STEPS // NBUF, body, 0)
    plsc.subcore_barrier()

    def writeback(r0):
        pltpu.sync_copy(acc_sh.at[pl.ds(r0, TCHUNK)], rows[0])
        pltpu.sync_copy(rows[0], out_hbm.at[c, pl.ds(r0, TCHUNK)])

    for rep in range(NTREP):
        ck = s + NS * rep

        @pl.when(ck < NTCH)
        def _():
            writeback(ck * TCHUNK)


BLK = 1000


def _dinv(dp_ref):
    deg = dp_ref[:, 0:1] + dp_ref[:, 1:2] - 1.0
    return lax.rsqrt(deg)


def _tc_in_body(x_ref, w_ref, dp_ref, o_ref):
    o_ref[...] = jnp.dot(x_ref[...], w_ref[...],
                         preferred_element_type=jnp.float32) * _dinv(dp_ref)


def _tc_mid_body(a0_ref, a1_ref, h_ref, dp_ref, b_ref, o_ref):
    dinv = _dinv(dp_ref)
    agg = a0_ref[...] + a1_ref[...] - h_ref[...]
    pre = agg * dinv + b_ref[...]
    o_ref[...] = jnp.maximum(pre, 0.0) * dinv


def _tc_out_body(a0_ref, a1_ref, h_ref, dp_ref, w_ref, b_ref, o_ref):
    dinv = _dinv(dp_ref)
    agg = (a0_ref[...] + a1_ref[...] - h_ref[...]) * dinv
    z = jnp.dot(agg, w_ref[...], preferred_element_type=jnp.float32) + b_ref[...]
    m = jnp.max(z, axis=1, keepdims=True)
    lse = jnp.log(jnp.sum(jnp.exp(z - m), axis=1, keepdims=True))
    o_ref[...] = z - m - lse


_tc_in = pl.pallas_call(
    _tc_in_body,
    grid=(N // BLK,),
    in_specs=[
        pl.BlockSpec((BLK, IN_DIM), lambda i: (i, 0)),
        pl.BlockSpec((IN_DIM, HID), lambda i: (0, 0)),
        pl.BlockSpec((BLK, 2), lambda i: (i, 0)),
    ],
    out_specs=pl.BlockSpec((BLK, HID), lambda i: (i, 0)),
    out_shape=jax.ShapeDtypeStruct((N, HID), jnp.float32),
)

_tc_mid = pl.pallas_call(
    _tc_mid_body,
    grid=(N // BLK,),
    in_specs=[
        pl.BlockSpec((BLK, HID), lambda i: (i, 0)),
        pl.BlockSpec((BLK, HID), lambda i: (i, 0)),
        pl.BlockSpec((BLK, HID), lambda i: (i, 0)),
        pl.BlockSpec((BLK, 2), lambda i: (i, 0)),
        pl.BlockSpec((1, HID), lambda i: (0, 0)),
    ],
    out_specs=pl.BlockSpec((BLK, HID), lambda i: (i, 0)),
    out_shape=jax.ShapeDtypeStruct((N, HID), jnp.float32),
)

_tc_out = pl.pallas_call(
    _tc_out_body,
    grid=(N // BLK,),
    in_specs=[
        pl.BlockSpec((BLK, HID), lambda i: (i, 0)),
        pl.BlockSpec((BLK, HID), lambda i: (i, 0)),
        pl.BlockSpec((BLK, HID), lambda i: (i, 0)),
        pl.BlockSpec((BLK, 2), lambda i: (i, 0)),
        pl.BlockSpec((HID, OUT_DIM), lambda i: (0, 0)),
        pl.BlockSpec((1, OUT_DIM), lambda i: (0, 0)),
    ],
    out_specs=pl.BlockSpec((BLK, OUT_DIM), lambda i: (i, 0)),
    out_shape=jax.ShapeDtypeStruct((N, OUT_DIM), jnp.float32),
)


@jax.jit
def kernel(x, edge_index, W1, b1, W2, b2):
    src = edge_index[0].reshape(E // CHUNK, CHUNK)
    dst = edge_index[1].reshape(E // CHUNK, CHUNK)

    degp = _deg_partials(dst).reshape(NC, N)
    dp = degp.T
    h1 = _tc_in(x, W1, dp)
    accp1 = _agg_partials(h1, src, dst)
    h2 = _tc_mid(accp1[0], accp1[1], h1, dp, b1.reshape(1, HID))
    accp2 = _agg_partials(h2, src, dst)
    return _tc_out(accp2[0], accp2[1], h2, dp, W2, b2.reshape(1, OUT_DIM))


# speedup vs baseline: 1.2358x; 1.2358x over previous
import functools

import jax
import jax.numpy as jnp
from jax import lax
from jax.experimental import pallas as pl
from jax.experimental.pallas import tpu as pltpu
from jax.experimental.pallas import tpu_sc as plsc

N = 10000
E = 320000
IN_DIM = 128
HID = 64
OUT_DIM = 128

NC = 2
NS = 16
CHUNK = 80
EDGES_PER_TILE = E // (NC * NS)
STEPS = EDGES_PER_TILE // CHUNK
NBUF = 5
RCHUNK = 400
NRCH = N // RCHUNK
NREP = -(-NRCH // NS)

_MESH = plsc.VectorSubcoreMesh(core_axis_name="c", subcore_axis_name="s")
_SC_PARAMS = pltpu.CompilerParams(use_tc_tiling_on_sc=False)


def _each_chunk(s, fn):
    for rep in range(NREP):
        ck = s + NS * rep

        @pl.when(ck < NRCH)
        def _():
            fn(ck * RCHUNK)


@functools.partial(
    pl.kernel,
    mesh=_MESH,
    compiler_params=_SC_PARAMS,
    out_type=jax.ShapeDtypeStruct((NC * N,), jnp.float32),
    scratch_types=[
        pltpu.VMEM((STEPS, CHUNK), jnp.int32),
        pltpu.VMEM((CHUNK,), jnp.float32),
        pltpu.VMEM((RCHUNK,), jnp.float32),
        pltpu.VMEM_SHARED((N,), jnp.float32),
    ],
)
def _deg_partials(dst_hbm, out_hbm, idx_v, ones_v, stage_v, deg_sh):
    c = lax.axis_index("c")
    s = lax.axis_index("s")
    tile_row = (c * NS + s) * STEPS
    pltpu.sync_copy(dst_hbm.at[pl.ds(tile_row, STEPS)], idx_v)
    for i in range(CHUNK // 16):
        ones_v[pl.ds(i * 16, 16)] = jnp.ones((16,), jnp.float32)
    for i in range(RCHUNK // 16):
        stage_v[pl.ds(i * 16, 16)] = jnp.ones((16,), jnp.float32)

    def init(r0):
        pltpu.sync_copy(stage_v, deg_sh.at[pl.ds(r0, RCHUNK)])

    _each_chunk(s, init)
    plsc.subcore_barrier()

    def body(i, carry):
        pltpu.sync_copy(ones_v, deg_sh.at[idx_v.at[i]], add=True)
        return carry

    lax.fori_loop(0, STEPS, body, 0)
    plsc.subcore_barrier()

    def writeback(r0):
        pltpu.sync_copy(deg_sh.at[pl.ds(r0, RCHUNK)], stage_v)
        pltpu.sync_copy(stage_v, out_hbm.at[pl.ds(c * N + r0, RCHUNK)])

    _each_chunk(s, writeback)


@functools.partial(
    pl.kernel,
    mesh=_MESH,
    compiler_params=_SC_PARAMS,
    out_type=jax.ShapeDtypeStruct((NC, N, HID), jnp.float32),
    scratch_types=[
        pltpu.VMEM((STEPS, CHUNK), jnp.int32),
        pltpu.VMEM((STEPS, CHUNK), jnp.int32),
        [pltpu.VMEM((CHUNK, HID), jnp.float32)] * NBUF,
        pltpu.VMEM((RCHUNK, HID), jnp.float32),
        pltpu.VMEM_SHARED((N, HID), jnp.float32),
        [pltpu.SemaphoreType.DMA] * NBUF,
    ],
)
def _agg_partials(h_hbm, src_hbm, dst_hbm, out_hbm,
                  sidx_v, didx_v, rows, stage_v, acc_sh, sems):
    c = lax.axis_index("c")
    s = lax.axis_index("s")
    def stage(r0):
        pltpu.sync_copy(h_hbm.at[pl.ds(r0, RCHUNK)], stage_v)
        pltpu.sync_copy(stage_v, acc_sh.at[pl.ds(r0, RCHUNK)])

    _each_chunk(s, stage)
    tile_row = (c * NS + s) * STEPS
    pltpu.sync_copy(src_hbm.at[pl.ds(tile_row, STEPS)], sidx_v)
    pltpu.sync_copy(dst_hbm.at[pl.ds(tile_row, STEPS)], didx_v)
    plsc.subcore_barrier()

    for b in range(NBUF):
        pltpu.async_copy(h_hbm.at[sidx_v.at[b]], rows[b], sems[b])

    def body(g, carry):
        j0 = g * NBUF
        for b in range(NBUF):
            jj = j0 + b
            pltpu.make_async_copy(h_hbm.at[sidx_v.at[0]], rows[b],
                                  sems[b]).wait()
            pltpu.sync_copy(rows[b], acc_sh.at[didx_v.at[jj]], add=True)

            @pl.when(jj + NBUF < STEPS)
            def _():
                pltpu.async_copy(h_hbm.at[sidx_v.at[jj + NBUF]],
                                 rows[b], sems[b])

        return carry

    lax.fori_loop(0, STEPS // NBUF, body, 0)
    plsc.subcore_barrier()

    def writeback(r0):
        pltpu.sync_copy(acc_sh.at[pl.ds(r0, RCHUNK)], stage_v)
        pltpu.sync_copy(stage_v, out_hbm.at[c, pl.ds(r0, RCHUNK)])

    _each_chunk(s, writeback)


BLK = 1000


def _dinv(dp_ref):
    deg = dp_ref[:, 0:1] + dp_ref[:, 1:2] - 1.0
    return lax.rsqrt(deg)


def _tc_in_body(x_ref, w_ref, dp_ref, o_ref):
    o_ref[...] = jnp.dot(x_ref[...], w_ref[...],
                         preferred_element_type=jnp.float32) * _dinv(dp_ref)


def _tc_mid_body(a0_ref, a1_ref, h_ref, dp_ref, b_ref, o_ref):
    dinv = _dinv(dp_ref)
    agg = a0_ref[...] + a1_ref[...] - h_ref[...]
    pre = agg * dinv + b_ref[...]
    o_ref[...] = jnp.maximum(pre, 0.0) * dinv


def _tc_out_body(a0_ref, a1_ref, h_ref, dp_ref, w_ref, b_ref, o_ref):
    dinv = _dinv(dp_ref)
    agg = (a0_ref[...] + a1_ref[...] - h_ref[...]) * dinv
    z = jnp.dot(agg, w_ref[...], preferred_element_type=jnp.float32) + b_ref[...]
    m = jnp.max(z, axis=1, keepdims=True)
    lse = jnp.log(jnp.sum(jnp.exp(z - m), axis=1, keepdims=True))
    o_ref[...] = z - m - lse


_tc_in = pl.pallas_call(
    _tc_in_body,
    grid=(N // BLK,),
    in_specs=[
        pl.BlockSpec((BLK, IN_DIM), lambda i: (i, 0)),
        pl.BlockSpec((IN_DIM, HID), lambda i: (0, 0)),
        pl.BlockSpec((BLK, 2), lambda i: (i, 0)),
    ],
    out_specs=pl.BlockSpec((BLK, HID), lambda i: (i, 0)),
    out_shape=jax.ShapeDtypeStruct((N, HID), jnp.float32),
)

_tc_mid = pl.pallas_call(
    _tc_mid_body,
    grid=(N // BLK,),
    in_specs=[
        pl.BlockSpec((BLK, HID), lambda i: (i, 0)),
        pl.BlockSpec((BLK, HID), lambda i: (i, 0)),
        pl.BlockSpec((BLK, HID), lambda i: (i, 0)),
        pl.BlockSpec((BLK, 2), lambda i: (i, 0)),
        pl.BlockSpec((1, HID), lambda i: (0, 0)),
    ],
    out_specs=pl.BlockSpec((BLK, HID), lambda i: (i, 0)),
    out_shape=jax.ShapeDtypeStruct((N, HID), jnp.float32),
)

_tc_out = pl.pallas_call(
    _tc_out_body,
    grid=(N // BLK,),
    in_specs=[
        pl.BlockSpec((BLK, HID), lambda i: (i, 0)),
        pl.BlockSpec((BLK, HID), lambda i: (i, 0)),
        pl.BlockSpec((BLK, HID), lambda i: (i, 0)),
        pl.BlockSpec((BLK, 2), lambda i: (i, 0)),
        pl.BlockSpec((HID, OUT_DIM), lambda i: (0, 0)),
        pl.BlockSpec((1, OUT_DIM), lambda i: (0, 0)),
    ],
    out_specs=pl.BlockSpec((BLK, OUT_DIM), lambda i: (i, 0)),
    out_shape=jax.ShapeDtypeStruct((N, OUT_DIM), jnp.float32),
)


@jax.jit
def kernel(x, edge_index, W1, b1, W2, b2):
    src = edge_index[0].reshape(E // CHUNK, CHUNK)
    dst = edge_index[1].reshape(E // CHUNK, CHUNK)

    degp = _deg_partials(dst).reshape(NC, N)
    dp = degp.T
    h1 = _tc_in(x, W1, dp)
    accp1 = _agg_partials(h1, src, dst)
    h2 = _tc_mid(accp1[0], accp1[1], h1, dp, b1.reshape(1, HID))
    accp2 = _agg_partials(h2, src, dst)
    return _tc_out(accp2[0], accp2[1], h2, dp, W2, b2.reshape(1, OUT_DIM))


# speedup vs baseline: 1.2365x; 1.0005x over previous
import functools

import jax
import jax.numpy as jnp
from jax import lax
from jax.experimental import pallas as pl
from jax.experimental.pallas import tpu as pltpu
from jax.experimental.pallas import tpu_sc as plsc

N = 10000
E = 320000
IN_DIM = 128
HID = 64
OUT_DIM = 128

NC = 2
NS = 16
CHUNK = 80
EDGES_PER_TILE = E // (NC * NS)
STEPS = EDGES_PER_TILE // CHUNK
NBUF = 5
RCHUNK = 400
NRCH = N // RCHUNK
NREP = -(-NRCH // NS)

_MESH = plsc.VectorSubcoreMesh(core_axis_name="c", subcore_axis_name="s")
_SC_PARAMS = pltpu.CompilerParams(use_tc_tiling_on_sc=False)


def _each_chunk(s, fn):
    for rep in range(NREP):
        ck = s + NS * rep

        @pl.when(ck < NRCH)
        def _():
            fn(ck * RCHUNK)


@functools.partial(
    pl.kernel,
    mesh=_MESH,
    compiler_params=_SC_PARAMS,
    out_type=jax.ShapeDtypeStruct((NC * N,), jnp.float32),
    scratch_types=[
        pltpu.VMEM((STEPS, CHUNK), jnp.int32),
        pltpu.VMEM((CHUNK,), jnp.float32),
        pltpu.VMEM((RCHUNK,), jnp.float32),
        pltpu.VMEM_SHARED((N,), jnp.float32),
    ],
)
def _deg_partials(dst_hbm, out_hbm, idx_v, ones_v, stage_v, deg_sh):
    c = lax.axis_index("c")
    s = lax.axis_index("s")
    tile_row = (c * NS + s) * STEPS
    pltpu.sync_copy(dst_hbm.at[pl.ds(tile_row, STEPS)], idx_v)
    for i in range(CHUNK // 16):
        ones_v[pl.ds(i * 16, 16)] = jnp.ones((16,), jnp.float32)
    for i in range(RCHUNK // 16):
        stage_v[pl.ds(i * 16, 16)] = jnp.ones((16,), jnp.float32)

    def init(r0):
        pltpu.sync_copy(stage_v, deg_sh.at[pl.ds(r0, RCHUNK)])

    _each_chunk(s, init)
    plsc.subcore_barrier()

    def body(i, carry):
        pltpu.sync_copy(ones_v, deg_sh.at[idx_v.at[i]], add=True)
        return carry

    lax.fori_loop(0, STEPS, body, 0)
    plsc.subcore_barrier()

    def writeback(r0):
        pltpu.sync_copy(deg_sh.at[pl.ds(r0, RCHUNK)], stage_v)
        pltpu.sync_copy(stage_v, out_hbm.at[pl.ds(c * N + r0, RCHUNK)])

    _each_chunk(s, writeback)


@functools.partial(
    pl.kernel,
    mesh=_MESH,
    compiler_params=_SC_PARAMS,
    out_type=jax.ShapeDtypeStruct((NC, N, HID), jnp.float32),
    scratch_types=[
        pltpu.VMEM((STEPS, CHUNK), jnp.int32),
        pltpu.VMEM((STEPS, CHUNK), jnp.int32),
        [pltpu.VMEM((CHUNK, HID), jnp.float32)] * NBUF,
        pltpu.VMEM((RCHUNK, HID), jnp.float32),
        pltpu.VMEM_SHARED((N, HID), jnp.float32),
        [pltpu.SemaphoreType.DMA] * NBUF,
    ],
)
def _agg_partials(h_hbm, src_hbm, dst_hbm, out_hbm,
                  sidx_v, didx_v, rows, stage_v, acc_sh, sems):
    c = lax.axis_index("c")
    s = lax.axis_index("s")
    def stage(r0):
        pltpu.sync_copy(h_hbm.at[pl.ds(r0, RCHUNK)], stage_v)
        pltpu.sync_copy(stage_v, acc_sh.at[pl.ds(r0, RCHUNK)])

    _each_chunk(s, stage)
    tile_row = (c * NS + s) * STEPS
    pltpu.sync_copy(src_hbm.at[pl.ds(tile_row, STEPS)], sidx_v)
    pltpu.sync_copy(dst_hbm.at[pl.ds(tile_row, STEPS)], didx_v)
    plsc.subcore_barrier()

    for b in range(NBUF):
        pltpu.async_copy(h_hbm.at[sidx_v.at[b]], rows[b], sems[b])

    def body(g, carry):
        j0 = g * NBUF
        for b in range(NBUF):
            jj = j0 + b
            pltpu.make_async_copy(h_hbm.at[sidx_v.at[0]], rows[b],
                                  sems[b]).wait()
            pltpu.sync_copy(rows[b], acc_sh.at[didx_v.at[jj]], add=True)

            @pl.when(jj + NBUF < STEPS)
            def _():
                pltpu.async_copy(h_hbm.at[sidx_v.at[jj + NBUF]],
                                 rows[b], sems[b])

        return carry

    lax.fori_loop(0, STEPS // NBUF, body, 0)
    plsc.subcore_barrier()

    def writeback(r0):
        pltpu.sync_copy(acc_sh.at[pl.ds(r0, RCHUNK)], stage_v)
        pltpu.sync_copy(stage_v, out_hbm.at[c, pl.ds(r0, RCHUNK)])

    _each_chunk(s, writeback)


BLK = 1000


def _dinv(dp_ref):
    deg = dp_ref[:, 0:1] + dp_ref[:, 1:2] - 1.0
    return lax.rsqrt(deg)


def _tc_mm1_body(x_ref, w_ref, o_ref):
    o_ref[...] = jnp.dot(x_ref[...], w_ref[...],
                         preferred_element_type=jnp.float32)


def _tc_scale_body(u_ref, dp_ref, o_ref):
    o_ref[...] = u_ref[...] * _dinv(dp_ref)


def _tc_mid_body(a0_ref, a1_ref, h_ref, dp_ref, b_ref, o_ref):
    dinv = _dinv(dp_ref)
    agg = a0_ref[...] + a1_ref[...] - h_ref[...]
    pre = agg * dinv + b_ref[...]
    o_ref[...] = jnp.maximum(pre, 0.0) * dinv


def _tc_out_body(a0_ref, a1_ref, h_ref, dp_ref, w_ref, b_ref, o_ref):
    dinv = _dinv(dp_ref)
    agg = (a0_ref[...] + a1_ref[...] - h_ref[...]) * dinv
    z = jnp.dot(agg, w_ref[...], preferred_element_type=jnp.float32) + b_ref[...]
    m = jnp.max(z, axis=1, keepdims=True)
    lse = jnp.log(jnp.sum(jnp.exp(z - m), axis=1, keepdims=True))
    o_ref[...] = z - m - lse


_tc_mm1 = pl.pallas_call(
    _tc_mm1_body,
    grid=(N // BLK,),
    in_specs=[
        pl.BlockSpec((BLK, IN_DIM), lambda i: (i, 0)),
        pl.BlockSpec((IN_DIM, HID), lambda i: (0, 0)),
    ],
    out_specs=pl.BlockSpec((BLK, HID), lambda i: (i, 0)),
    out_shape=jax.ShapeDtypeStruct((N, HID), jnp.float32),
)

_tc_scale = pl.pallas_call(
    _tc_scale_body,
    grid=(N // BLK,),
    in_specs=[
        pl.BlockSpec((BLK, HID), lambda i: (i, 0)),
        pl.BlockSpec((BLK, 2), lambda i: (i, 0)),
    ],
    out_specs=pl.BlockSpec((BLK, HID), lambda i: (i, 0)),
    out_shape=jax.ShapeDtypeStruct((N, HID), jnp.float32),
)

_tc_mid = pl.pallas_call(
    _tc_mid_body,
    grid=(N // BLK,),
    in_specs=[
        pl.BlockSpec((BLK, HID), lambda i: (i, 0)),
        pl.BlockSpec((BLK, HID), lambda i: (i, 0)),
        pl.BlockSpec((BLK, HID), lambda i: (i, 0)),
        pl.BlockSpec((BLK, 2), lambda i: (i, 0)),
        pl.BlockSpec((1, HID), lambda i: (0, 0)),
    ],
    out_specs=pl.BlockSpec((BLK, HID), lambda i: (i, 0)),
    out_shape=jax.ShapeDtypeStruct((N, HID), jnp.float32),
)

_tc_out = pl.pallas_call(
    _tc_out_body,
    grid=(N // BLK,),
    in_specs=[
        pl.BlockSpec((BLK, HID), lambda i: (i, 0)),
        pl.BlockSpec((BLK, HID), lambda i: (i, 0)),
        pl.BlockSpec((BLK, HID), lambda i: (i, 0)),
        pl.BlockSpec((BLK, 2), lambda i: (i, 0)),
        pl.BlockSpec((HID, OUT_DIM), lambda i: (0, 0)),
        pl.BlockSpec((1, OUT_DIM), lambda i: (0, 0)),
    ],
    out_specs=pl.BlockSpec((BLK, OUT_DIM), lambda i: (i, 0)),
    out_shape=jax.ShapeDtypeStruct((N, OUT_DIM), jnp.float32),
)


@jax.jit
def kernel(x, edge_index, W1, b1, W2, b2):
    src = edge_index[0].reshape(E // CHUNK, CHUNK)
    dst = edge_index[1].reshape(E // CHUNK, CHUNK)

    u = _tc_mm1(x, W1)
    degp = _deg_partials(dst).reshape(NC, N)
    dp = degp.T
    h1 = _tc_scale(u, dp)
    accp1 = _agg_partials(h1, src, dst)
    h2 = _tc_mid(accp1[0], accp1[1], h1, dp, b1.reshape(1, HID))
    accp2 = _agg_partials(h2, src, dst)
    return _tc_out(accp2[0], accp2[1], h2, dp, W2, b2.reshape(1, OUT_DIM))


# speedup vs baseline: 1.3866x; 1.1214x over previous
import functools

import jax
import jax.numpy as jnp
from jax import lax
from jax.experimental import pallas as pl
from jax.experimental.pallas import tpu as pltpu
from jax.experimental.pallas import tpu_sc as plsc

N = 10000
E = 320000
IN_DIM = 128
HID = 64
OUT_DIM = 128

NC = 2
NS = 16
CHUNK = 80
EDGES_PER_TILE = E // (NC * NS)
STEPS = EDGES_PER_TILE // CHUNK
NBUF = 5
RCHUNK = 400
NRCH = N // RCHUNK
NREP = -(-NRCH // NS)

_MESH = plsc.VectorSubcoreMesh(core_axis_name="c", subcore_axis_name="s")
_SC_PARAMS = pltpu.CompilerParams(use_tc_tiling_on_sc=False)


def _each_chunk(s, fn):
    for rep in range(NREP):
        ck = s + NS * rep

        @pl.when(ck < NRCH)
        def _():
            fn(ck * RCHUNK)


@functools.partial(
    pl.kernel,
    mesh=_MESH,
    compiler_params=_SC_PARAMS,
    out_type=jax.ShapeDtypeStruct((NC * N,), jnp.float32),
    scratch_types=[
        pltpu.VMEM((STEPS, CHUNK), jnp.int32),
        pltpu.VMEM((CHUNK,), jnp.float32),
        pltpu.VMEM((RCHUNK,), jnp.float32),
        pltpu.VMEM_SHARED((N,), jnp.float32),
    ],
)
def _deg_partials(ei_hbm, out_hbm, idx_v, ones_v, stage_v, deg_sh):
    c = lax.axis_index("c")
    s = lax.axis_index("s")
    tile_row = (c * NS + s) * STEPS
    pltpu.sync_copy(ei_hbm.at[1, pl.ds(tile_row, STEPS)], idx_v)
    for i in range(CHUNK // 16):
        ones_v[pl.ds(i * 16, 16)] = jnp.ones((16,), jnp.float32)
    for i in range(RCHUNK // 16):
        stage_v[pl.ds(i * 16, 16)] = jnp.ones((16,), jnp.float32)

    def init(r0):
        pltpu.sync_copy(stage_v, deg_sh.at[pl.ds(r0, RCHUNK)])

    _each_chunk(s, init)
    plsc.subcore_barrier()

    def body(i, carry):
        pltpu.sync_copy(ones_v, deg_sh.at[idx_v.at[i]], add=True)
        return carry

    lax.fori_loop(0, STEPS, body, 0)
    plsc.subcore_barrier()

    def writeback(r0):
        pltpu.sync_copy(deg_sh.at[pl.ds(r0, RCHUNK)], stage_v)
        pltpu.sync_copy(stage_v, out_hbm.at[pl.ds(c * N + r0, RCHUNK)])

    _each_chunk(s, writeback)


@functools.partial(
    pl.kernel,
    mesh=_MESH,
    compiler_params=_SC_PARAMS,
    out_type=jax.ShapeDtypeStruct((NC, N, HID), jnp.float32),
    scratch_types=[
        pltpu.VMEM((STEPS, CHUNK), jnp.int32),
        pltpu.VMEM((STEPS, CHUNK), jnp.int32),
        [pltpu.VMEM((CHUNK, HID), jnp.float32)] * NBUF,
        pltpu.VMEM((RCHUNK, HID), jnp.float32),
        pltpu.VMEM_SHARED((N, HID), jnp.float32),
        [pltpu.SemaphoreType.DMA] * NBUF,
    ],
)
def _agg_partials(h_hbm, ei_hbm, out_hbm,
                  sidx_v, didx_v, rows, stage_v, acc_sh, sems):
    c = lax.axis_index("c")
    s = lax.axis_index("s")
    def stage(r0):
        pltpu.sync_copy(h_hbm.at[pl.ds(r0, RCHUNK)], stage_v)
        pltpu.sync_copy(stage_v, acc_sh.at[pl.ds(r0, RCHUNK)])

    _each_chunk(s, stage)
    tile_row = (c * NS + s) * STEPS
    pltpu.sync_copy(ei_hbm.at[0, pl.ds(tile_row, STEPS)], sidx_v)
    pltpu.sync_copy(ei_hbm.at[1, pl.ds(tile_row, STEPS)], didx_v)
    plsc.subcore_barrier()

    for b in range(NBUF):
        pltpu.async_copy(h_hbm.at[sidx_v.at[b]], rows[b], sems[b])

    def body(g, carry):
        j0 = g * NBUF
        for b in range(NBUF):
            jj = j0 + b
            pltpu.make_async_copy(h_hbm.at[sidx_v.at[0]], rows[b],
                                  sems[b]).wait()
            pltpu.sync_copy(rows[b], acc_sh.at[didx_v.at[jj]], add=True)

            @pl.when(jj + NBUF < STEPS)
            def _():
                pltpu.async_copy(h_hbm.at[sidx_v.at[jj + NBUF]],
                                 rows[b], sems[b])

        return carry

    lax.fori_loop(0, STEPS // NBUF, body, 0)
    plsc.subcore_barrier()

    def writeback(r0):
        pltpu.sync_copy(acc_sh.at[pl.ds(r0, RCHUNK)], stage_v)
        pltpu.sync_copy(stage_v, out_hbm.at[c, pl.ds(r0, RCHUNK)])

    _each_chunk(s, writeback)


BLK = 1000


def _dinv(dp_ref):
    deg = dp_ref[:, 0:1] + dp_ref[:, 1:2] - 1.0
    return lax.rsqrt(deg)


def _tc_mm1_body(x_ref, w_ref, o_ref):
    o_ref[...] = jnp.dot(x_ref[...], w_ref[...],
                         preferred_element_type=jnp.float32)


def _tc_scale_body(u_ref, dp_ref, o_ref):
    o_ref[...] = u_ref[...] * _dinv(dp_ref)


def _tc_mid_body(ap_ref, h_ref, dp_ref, b_ref, o_ref):
    dinv = _dinv(dp_ref)
    agg = ap_ref[0] + ap_ref[1] - h_ref[...]
    pre = agg * dinv + b_ref[...]
    o_ref[...] = jnp.maximum(pre, 0.0) * dinv


def _tc_out_body(ap_ref, h_ref, dp_ref, w_ref, b_ref, o_ref):
    dinv = _dinv(dp_ref)
    agg = (ap_ref[0] + ap_ref[1] - h_ref[...]) * dinv
    z = jnp.dot(agg, w_ref[...], preferred_element_type=jnp.float32) + b_ref[...]
    m = jnp.max(z, axis=1, keepdims=True)
    lse = jnp.log(jnp.sum(jnp.exp(z - m), axis=1, keepdims=True))
    o_ref[...] = z - m - lse


_tc_mm1 = pl.pallas_call(
    _tc_mm1_body,
    grid=(N // BLK,),
    in_specs=[
        pl.BlockSpec((BLK, IN_DIM), lambda i: (i, 0)),
        pl.BlockSpec((IN_DIM, HID), lambda i: (0, 0)),
    ],
    out_specs=pl.BlockSpec((BLK, HID), lambda i: (i, 0)),
    out_shape=jax.ShapeDtypeStruct((N, HID), jnp.float32),
)

_tc_scale = pl.pallas_call(
    _tc_scale_body,
    grid=(N // BLK,),
    in_specs=[
        pl.BlockSpec((BLK, HID), lambda i: (i, 0)),
        pl.BlockSpec((BLK, 2), lambda i: (i, 0)),
    ],
    out_specs=pl.BlockSpec((BLK, HID), lambda i: (i, 0)),
    out_shape=jax.ShapeDtypeStruct((N, HID), jnp.float32),
)

_tc_mid = pl.pallas_call(
    _tc_mid_body,
    grid=(N // BLK,),
    in_specs=[
        pl.BlockSpec((NC, BLK, HID), lambda i: (0, i, 0)),
        pl.BlockSpec((BLK, HID), lambda i: (i, 0)),
        pl.BlockSpec((BLK, 2), lambda i: (i, 0)),
        pl.BlockSpec((1, HID), lambda i: (0, 0)),
    ],
    out_specs=pl.BlockSpec((BLK, HID), lambda i: (i, 0)),
    out_shape=jax.ShapeDtypeStruct((N, HID), jnp.float32),
)

_tc_out = pl.pallas_call(
    _tc_out_body,
    grid=(N // BLK,),
    in_specs=[
        pl.BlockSpec((NC, BLK, HID), lambda i: (0, i, 0)),
        pl.BlockSpec((BLK, HID), lambda i: (i, 0)),
        pl.BlockSpec((BLK, 2), lambda i: (i, 0)),
        pl.BlockSpec((HID, OUT_DIM), lambda i: (0, 0)),
        pl.BlockSpec((1, OUT_DIM), lambda i: (0, 0)),
    ],
    out_specs=pl.BlockSpec((BLK, OUT_DIM), lambda i: (i, 0)),
    out_shape=jax.ShapeDtypeStruct((N, OUT_DIM), jnp.float32),
)


@jax.jit
def kernel(x, edge_index, W1, b1, W2, b2):
    ei = edge_index.reshape(2, E // CHUNK, CHUNK)

    u = _tc_mm1(x, W1)
    degp = _deg_partials(ei).reshape(NC, N)
    dp = degp.T
    h1 = _tc_scale(u, dp)
    accp1 = _agg_partials(h1, ei)
    h2 = _tc_mid(accp1, h1, dp, b1.reshape(1, HID))
    accp2 = _agg_partials(h2, ei)
    return _tc_out(accp2, h2, dp, W2, b2.reshape(1, OUT_DIM))


# speedup vs baseline: 1.4258x; 1.0282x over previous
import functools

import jax
import jax.numpy as jnp
from jax import lax
from jax.experimental import pallas as pl
from jax.experimental.pallas import tpu as pltpu
from jax.experimental.pallas import tpu_sc as plsc

N = 10000
E = 320000
IN_DIM = 128
HID = 64
OUT_DIM = 128

NC = 2
NS = 16
CHUNK = 80
EDGES_PER_TILE = E // (NC * NS)
STEPS = EDGES_PER_TILE // CHUNK
NBUF = 5
RCHUNK = 400
NRCH = N // RCHUNK
NREP = -(-NRCH // NS)

_MESH = plsc.VectorSubcoreMesh(core_axis_name="c", subcore_axis_name="s")
_SC_PARAMS = pltpu.CompilerParams(use_tc_tiling_on_sc=False)


def _each_chunk(s, fn):
    for rep in range(NREP):
        ck = s + NS * rep

        @pl.when(ck < NRCH)
        def _():
            fn(ck * RCHUNK)


@functools.partial(
    pl.kernel,
    mesh=_MESH,
    compiler_params=_SC_PARAMS,
    out_type=jax.ShapeDtypeStruct((NC * N,), jnp.float32),
    scratch_types=[
        pltpu.VMEM((STEPS, CHUNK), jnp.int32),
        pltpu.VMEM((CHUNK,), jnp.float32),
        pltpu.VMEM((RCHUNK,), jnp.float32),
        pltpu.VMEM_SHARED((N,), jnp.float32),
    ],
)
def _deg_partials(ei_hbm, out_hbm, idx_v, ones_v, stage_v, deg_sh):
    c = lax.axis_index("c")
    s = lax.axis_index("s")
    tile_row = (c * NS + s) * STEPS
    pltpu.sync_copy(ei_hbm.at[1, pl.ds(tile_row, STEPS)], idx_v)
    for i in range(CHUNK // 16):
        ones_v[pl.ds(i * 16, 16)] = jnp.ones((16,), jnp.float32)
    for i in range(RCHUNK // 16):
        stage_v[pl.ds(i * 16, 16)] = jnp.ones((16,), jnp.float32)

    def init(r0):
        pltpu.sync_copy(stage_v, deg_sh.at[pl.ds(r0, RCHUNK)])

    _each_chunk(s, init)
    plsc.subcore_barrier()

    def body(i, carry):
        pltpu.sync_copy(ones_v, deg_sh.at[idx_v.at[i]], add=True)
        return carry

    lax.fori_loop(0, STEPS, body, 0)
    plsc.subcore_barrier()

    def writeback(r0):
        pltpu.sync_copy(deg_sh.at[pl.ds(r0, RCHUNK)], stage_v)
        pltpu.sync_copy(stage_v, out_hbm.at[pl.ds(c * N + r0, RCHUNK)])

    _each_chunk(s, writeback)


@functools.partial(
    pl.kernel,
    mesh=_MESH,
    compiler_params=_SC_PARAMS,
    out_type=jax.ShapeDtypeStruct((NC, N, HID), jnp.float32),
    scratch_types=[
        pltpu.VMEM((STEPS, CHUNK), jnp.int32),
        pltpu.VMEM((STEPS, CHUNK), jnp.int32),
        [pltpu.VMEM((CHUNK, HID), jnp.float32)] * NBUF,
        pltpu.VMEM((RCHUNK, HID), jnp.float32),
        pltpu.VMEM_SHARED((N, HID), jnp.float32),
        [pltpu.SemaphoreType.DMA] * NBUF,
    ],
)
def _agg_partials(h_hbm, ei_hbm, out_hbm,
                  sidx_v, didx_v, rows, stage_v, acc_sh, sems):
    c = lax.axis_index("c")
    s = lax.axis_index("s")
    def stage(r0):
        pltpu.sync_copy(h_hbm.at[pl.ds(r0, RCHUNK)], stage_v)
        pltpu.sync_copy(stage_v, acc_sh.at[pl.ds(r0, RCHUNK)])

    _each_chunk(s, stage)
    tile_row = (c * NS + s) * STEPS
    pltpu.sync_copy(ei_hbm.at[0, pl.ds(tile_row, STEPS)], sidx_v)
    pltpu.sync_copy(ei_hbm.at[1, pl.ds(tile_row, STEPS)], didx_v)
    plsc.subcore_barrier()

    for b in range(NBUF):
        pltpu.async_copy(h_hbm.at[sidx_v.at[b]], rows[b], sems[b])

    def body(g, carry):
        j0 = g * NBUF
        for b in range(NBUF):
            jj = j0 + b
            pltpu.make_async_copy(h_hbm.at[sidx_v.at[0]], rows[b],
                                  sems[b]).wait()
            pltpu.sync_copy(rows[b], acc_sh.at[didx_v.at[jj]], add=True)

            @pl.when(jj + NBUF < STEPS)
            def _():
                pltpu.async_copy(h_hbm.at[sidx_v.at[jj + NBUF]],
                                 rows[b], sems[b])

        return carry

    lax.fori_loop(0, STEPS // NBUF, body, 0)
    plsc.subcore_barrier()

    def writeback(r0):
        pltpu.sync_copy(acc_sh.at[pl.ds(r0, RCHUNK)], stage_v)
        pltpu.sync_copy(stage_v, out_hbm.at[c, pl.ds(r0, RCHUNK)])

    _each_chunk(s, writeback)


BLK = 2000


def _dinv(dp_ref):
    deg = dp_ref[:, 0:1] + dp_ref[:, 1:2] - 1.0
    return lax.rsqrt(deg)


def _tc_mm1_body(x_ref, w_ref, o_ref):
    o_ref[...] = jnp.dot(x_ref[...], w_ref[...],
                         preferred_element_type=jnp.float32)


def _tc_scale_body(u_ref, dp_ref, o_ref):
    o_ref[...] = u_ref[...] * _dinv(dp_ref)


def _tc_mid_body(ap_ref, h_ref, dp_ref, b_ref, o_ref):
    dinv = _dinv(dp_ref)
    agg = ap_ref[0] + ap_ref[1] - h_ref[...]
    pre = agg * dinv + b_ref[...]
    o_ref[...] = jnp.maximum(pre, 0.0) * dinv


def _tc_out_body(ap_ref, h_ref, dp_ref, w_ref, b_ref, o_ref):
    dinv = _dinv(dp_ref)
    agg = (ap_ref[0] + ap_ref[1] - h_ref[...]) * dinv
    z = jnp.dot(agg, w_ref[...], preferred_element_type=jnp.float32) + b_ref[...]
    m = jnp.max(z, axis=1, keepdims=True)
    lse = jnp.log(jnp.sum(jnp.exp(z - m), axis=1, keepdims=True))
    o_ref[...] = z - m - lse


_tc_mm1 = pl.pallas_call(
    _tc_mm1_body,
    grid=(N // BLK,),
    in_specs=[
        pl.BlockSpec((BLK, IN_DIM), lambda i: (i, 0)),
        pl.BlockSpec((IN_DIM, HID), lambda i: (0, 0)),
    ],
    out_specs=pl.BlockSpec((BLK, HID), lambda i: (i, 0)),
    out_shape=jax.ShapeDtypeStruct((N, HID), jnp.float32),
)

_tc_scale = pl.pallas_call(
    _tc_scale_body,
    grid=(N // BLK,),
    in_specs=[
        pl.BlockSpec((BLK, HID), lambda i: (i, 0)),
        pl.BlockSpec((BLK, 2), lambda i: (i, 0)),
    ],
    out_specs=pl.BlockSpec((BLK, HID), lambda i: (i, 0)),
    out_shape=jax.ShapeDtypeStruct((N, HID), jnp.float32),
)

_tc_mid = pl.pallas_call(
    _tc_mid_body,
    grid=(N // BLK,),
    in_specs=[
        pl.BlockSpec((NC, BLK, HID), lambda i: (0, i, 0)),
        pl.BlockSpec((BLK, HID), lambda i: (i, 0)),
        pl.BlockSpec((BLK, 2), lambda i: (i, 0)),
        pl.BlockSpec((1, HID), lambda i: (0, 0)),
    ],
    out_specs=pl.BlockSpec((BLK, HID), lambda i: (i, 0)),
    out_shape=jax.ShapeDtypeStruct((N, HID), jnp.float32),
)

_tc_out = pl.pallas_call(
    _tc_out_body,
    grid=(N // BLK,),
    in_specs=[
        pl.BlockSpec((NC, BLK, HID), lambda i: (0, i, 0)),
        pl.BlockSpec((BLK, HID), lambda i: (i, 0)),
        pl.BlockSpec((BLK, 2), lambda i: (i, 0)),
        pl.BlockSpec((HID, OUT_DIM), lambda i: (0, 0)),
        pl.BlockSpec((1, OUT_DIM), lambda i: (0, 0)),
    ],
    out_specs=pl.BlockSpec((BLK, OUT_DIM), lambda i: (i, 0)),
    out_shape=jax.ShapeDtypeStruct((N, OUT_DIM), jnp.float32),
)


@jax.jit
def kernel(x, edge_index, W1, b1, W2, b2):
    ei = edge_index.reshape(2, E // CHUNK, CHUNK)

    u = _tc_mm1(x, W1)
    degp = _deg_partials(ei).reshape(NC, N)
    dp = degp.T
    h1 = _tc_scale(u, dp)
    accp1 = _agg_partials(h1, ei)
    h2 = _tc_mid(accp1, h1, dp, b1.reshape(1, HID))
    accp2 = _agg_partials(h2, ei)
    return _tc_out(accp2, h2, dp, W2, b2.reshape(1, OUT_DIM))
